# baseline (speedup 1.0000x reference)
import functools

import numpy as np
import jax
import jax.numpy as jnp
from jax import lax
from jax.experimental import pallas as pl
from jax.experimental.pallas import tpu as pltpu
from jax.experimental.pallas import tpu_sc as plsc


def _np_threefry2x32(k1, k2, x0, x1):
    u32 = np.uint32
    rotl = lambda x, r: u32((u32(x) << u32(r)) | (u32(x) >> u32(32 - r)))
    ks = [u32(k1), u32(k2), u32(u32(k1) ^ u32(k2) ^ u32(0x1BD11BDA))]
    x = [u32(x0 + ks[0]), u32(x1 + ks[1])]
    rots = ((13, 15, 26, 6), (17, 29, 16, 24))
    old = np.seterr(over="ignore")
    for i in range(5):
        for r in rots[i % 2]:
            x[0] = u32(x[0] + x[1])
            x[1] = u32(rotl(x[1], r) ^ x[0])
        x[0] = u32(x[0] + ks[(i + 1) % 3])
        x[1] = u32(x[1] + ks[(i + 2) % 3] + u32(i + 1))
    np.seterr(**old)
    return int(x[0]), int(x[1])


_BASE_K1, _BASE_K2 = 0, 42
_NOISE_KEY = _np_threefry2x32(_BASE_K1, _BASE_K2, 0, 0)
_CAT_KEY = _np_threefry2x32(_BASE_K1, _BASE_K2, 0, 1)

_TINY = np.float32(np.finfo(np.float32).tiny)
_U_LO = np.float32(np.nextafter(np.float32(-1.0), np.float32(0.0)))
_U_SPAN = np.float32(np.float32(1.0) - _U_LO)
_G_SPAN = np.float32(np.float32(1.0) - _TINY)
_SQRT2 = np.float32(np.sqrt(2.0))


def _tf2x32(k1, k2, x0, x1):
    ks0 = jnp.uint32(k1)
    ks1 = jnp.uint32(k2)
    ks2 = jnp.uint32(k1 ^ k2 ^ 0x1BD11BDA)
    kss = (ks0, ks1, ks2)

    def rounds(x0, x1, rs):
        for r in rs:
            x0 = x0 + x1
            x1 = (x1 << jnp.uint32(r)) | (x1 >> jnp.uint32(32 - r))
            x1 = x1 ^ x0
        return x0, x1

    rots = ((13, 15, 26, 6), (17, 29, 16, 24))
    x0 = x0 + ks0
    x1 = x1 + ks1
    for i in range(5):
        x0, x1 = rounds(x0, x1, rots[i % 2])
        x0 = x0 + kss[(i + 1) % 3]
        x1 = x1 + kss[(i + 2) % 3] + jnp.uint32(i + 1)
    return x0, x1


def _bits_to_unit(bits):
    fb = (bits >> jnp.uint32(9)) | jnp.uint32(0x3F800000)
    return lax.bitcast_convert_type(fb, jnp.float32) - jnp.float32(1.0)


def _erfinv_f32(x):
    w = -jnp.log(jnp.float32(1.0) - x * x)
    wc = w - jnp.float32(2.5)
    p = jnp.float32(2.81022636e-08)
    for c in (3.43273939e-07, -3.5233877e-06, -4.39150654e-06, 0.00021858087,
              -0.00125372503, -0.00417768164, 0.246640727, 1.50140941):
        p = jnp.float32(c) + p * wc
    wt = jnp.sqrt(w) - jnp.float32(3.0)
    q = jnp.float32(-0.000200214257)
    for c in (0.000100950558, 0.00134934322, -0.00367342844, 0.00573950773,
              -0.0076224613, 0.00943887047, 1.00167406, 2.83297682):
        q = jnp.float32(c) + q * wt
    return jnp.where(w < jnp.float32(5.0), p, q) * x


def _noise_body(out_ref):
    rb = pl.program_id(0)
    rows, lanes = out_ref.shape
    base = jnp.uint32(rb * rows * lanes)
    lo = (base
          + lax.broadcasted_iota(jnp.uint32, (rows, lanes), 0) * jnp.uint32(lanes)
          + lax.broadcasted_iota(jnp.uint32, (rows, lanes), 1))
    b0, b1 = _tf2x32(_NOISE_KEY[0], _NOISE_KEY[1], jnp.uint32(0), lo)
    f = _bits_to_unit(b0 ^ b1)
    u = jnp.maximum(_U_LO, f * _U_SPAN + _U_LO)
    out_ref[...] = (_SQRT2 * _erfinv_f32(u)) * jnp.float32(0.1)


def _make_noise(total_rows, lanes=128, block_rows=1024):
    block_rows = min(block_rows, total_rows)
    return pl.pallas_call(
        _noise_body,
        grid=(total_rows // block_rows,),
        out_specs=pl.BlockSpec((block_rows, lanes), lambda rb: (rb, 0)),
        out_shape=jax.ShapeDtypeStruct((total_rows, lanes), jnp.float32),
    )()


def _predict_body(sp_ref, noise_ref, lwp_ref, obs_ref, ctrl_ref, a_ref, b_ref,
                  c_ref, pred_ref, lw_ref):
    sp = sp_ref[0]
    cb = jnp.dot(ctrl_ref[0], b_ref[...],
                 preferred_element_type=jnp.float32)
    pred = (jnp.dot(sp, a_ref[...], preferred_element_type=jnp.float32)
            + cb + noise_ref[0])
    diff = jnp.dot(pred, c_ref[...],
                   preferred_element_type=jnp.float32) - obs_ref[0]
    meas = jnp.float32(-0.5) * jnp.sum(diff * diff, axis=1)
    pred_ref[0] = pred
    lw_ref[0, 0, :] = lwp_ref[0, 0] + meas


def _run_predict(states_prev, noise, log_weights_prev, observations, controls,
                 A, B, C, mb=2048):
    mb = min(mb, states_prev.shape[1])
    n, m, d = states_prev.shape
    do = observations.shape[1]
    dc = controls.shape[1]
    nb = m // mb
    lwp3 = log_weights_prev.reshape(n * nb, 1, mb)
    obs3 = observations.reshape(n, 1, do)
    ctrl3 = controls.reshape(n, 1, dc)
    pred, lw3 = pl.pallas_call(
        _predict_body,
        grid=(n, nb),
        in_specs=[
            pl.BlockSpec((1, mb, d), lambda j, b: (j, b, 0)),
            pl.BlockSpec((1, mb, d), lambda j, b: (j, b, 0)),
            pl.BlockSpec((1, 1, mb), lambda j, b, _nb=nb: (j * _nb + b, 0, 0)),
            pl.BlockSpec((1, 1, do), lambda j, b: (j, 0, 0)),
            pl.BlockSpec((1, 1, dc), lambda j, b: (j, 0, 0)),
            pl.BlockSpec((d, d), lambda j, b: (0, 0)),
            pl.BlockSpec((dc, d), lambda j, b: (0, 0)),
            pl.BlockSpec((d, do), lambda j, b: (0, 0)),
        ],
        out_specs=[
            pl.BlockSpec((1, mb, d), lambda j, b: (j, b, 0)),
            pl.BlockSpec((1, 1, mb), lambda j, b, _nb=nb: (j * _nb + b, 0, 0)),
        ],
        out_shape=[
            jax.ShapeDtypeStruct((n, m, d), jnp.float32),
            jax.ShapeDtypeStruct((n * nb, 1, mb), jnp.float32),
        ],
    )(states_prev, noise, lwp3, obs3, ctrl3, A, B, C)
    return pred, lw3.reshape(n, m)


def _stats_body(lw_ref, pred_ref, best_ref):
    lw = lw_ref[0]
    m0 = jnp.max(lw)
    e = jnp.exp(lw - m0)
    s = jnp.sum(e)
    acc = jnp.dot(e, pred_ref[0], preferred_element_type=jnp.float32)
    best_ref[0] = acc / s


def _run_stats(lw, states_pred):
    n, m, d = states_pred.shape
    best3 = pl.pallas_call(
        _stats_body,
        grid=(n,),
        in_specs=[
            pl.BlockSpec((1, 1, m), lambda j: (j, 0, 0)),
            pl.BlockSpec((1, m, d), lambda j: (j, 0, 0)),
        ],
        out_specs=pl.BlockSpec((1, 1, d), lambda j: (j, 0, 0)),
        out_shape=jax.ShapeDtypeStruct((n, 1, d), jnp.float32),
    )(lw.reshape(n, 1, m), states_pred)
    return best3.reshape(n, d)


def _sample_body(lw_ref, idx_ref, *, n, m, sb):
    j = pl.program_id(0)
    ib = pl.program_id(1)
    nm_shift = (n * m).bit_length() - 1
    m_shift = m.bit_length() - 1
    i_base = ib * sb
    hi = jnp.uint32(i_base >> (32 - nm_shift))
    lane_u = lax.broadcasted_iota(jnp.uint32, (8, 128), 1)
    lane_i = lax.broadcasted_iota(jnp.int32, (8, 128), 1)
    sub_u = lax.broadcasted_iota(jnp.uint32, (8, 128), 0)
    lo_ij = ((jnp.uint32(i_base) + sub_u) << jnp.uint32(nm_shift)) | (
        jnp.uint32(j) << jnp.uint32(m_shift))

    def isub_body(i_sub, _):
        lo0 = lo_ij + (jnp.uint32(i_sub * 8) << jnp.uint32(nm_shift))

        def kb_body(kb, carry):
            best_v, best_k = carry
            k0 = kb * 128
            lo = lo0 + jnp.uint32(k0) + lane_u
            b0, b1 = _tf2x32(_CAT_KEY[0], _CAT_KEY[1], hi, lo)
            f = _bits_to_unit(b0 ^ b1)
            u = jnp.maximum(_TINY, f * _G_SPAN + _TINY)
            g = -jnp.log(-jnp.log(u))
            lvec = lw_ref[0, 0, pl.ds(k0, 128)]
            tot = g + jnp.broadcast_to(lvec[None, :], (8, 128))
            kvec = jnp.int32(k0) + lane_i
            upd = tot > best_v
            return (jnp.where(upd, tot, best_v), jnp.where(upd, kvec, best_k))

        best_v, best_k = lax.fori_loop(
            0, m // 128, kb_body,
            (jnp.full((8, 128), -jnp.inf, jnp.float32),
             jnp.zeros((8, 128), jnp.int32)))
        vmax = jnp.max(best_v, axis=1, keepdims=True)
        kmin = jnp.min(jnp.where(best_v == vmax, best_k, jnp.int32(m)), axis=1)
        idx_ref[0, i_sub, :] = kmin + j * m
        return 0

    lax.fori_loop(0, sb // 8, isub_body, 0)


def _run_sample(lw, sb=256):
    n, m = lw.shape
    nib = m // sb
    body = functools.partial(_sample_body, n=n, m=m, sb=sb)
    idx3 = pl.pallas_call(
        body,
        grid=(n, nib),
        in_specs=[pl.BlockSpec((1, 1, m), lambda j, ib: (j, 0, 0))],
        out_specs=pl.BlockSpec((1, sb // 8, 8),
                               lambda j, ib, _nib=nib: (j * _nib + ib, 0, 0)),
        out_shape=jax.ShapeDtypeStruct((n * nib, sb // 8, 8), jnp.int32),
    )(lw.reshape(n, 1, m))
    return idx3.reshape(n, m)


def _run_gather(table8, idx2d, d, chunk=128):
    nrows, _ = table8.shape
    nm = nrows * (128 // d)
    ppr = 128 // d
    info = plsc.get_sparse_core_info()
    nw = info.num_cores * info.num_subcores
    n_chunks = nm // nw // chunk
    orpc = chunk * d // 128
    mesh = plsc.VectorSubcoreMesh(core_axis_name="c", subcore_axis_name="s")

    @functools.partial(
        pl.kernel,
        mesh=mesh,
        out_type=jax.ShapeDtypeStruct((nm * d // 128, 128), jnp.float32),
        scratch_types=[
            pltpu.VMEM((n_chunks, chunk), jnp.int32),
            pltpu.VMEM((chunk,), jnp.int32),
            pltpu.VMEM((chunk, 128), jnp.float32),
            pltpu.VMEM((orpc, 128), jnp.float32),
            pltpu.SemaphoreType.DMA,
        ],
    )
    def k(table_hbm, idx_hbm, out_hbm, idx_v, rows_v, buf_v, outb_v, sem):
        wid = lax.axis_index("s") * info.num_cores + lax.axis_index("c")
        c0 = wid * n_chunks
        pltpu.sync_copy(idx_hbm.at[pl.ds(c0, n_chunks)], idx_v)
        iota16 = lax.iota(jnp.int32, 16)

        def chunk_body(c, _):
            for g in range(chunk // 16):
                iv = idx_v[c, pl.ds(g * 16, 16)]
                rows_v[pl.ds(g * 16, 16)] = lax.shift_right_logical(iv, 3)
            pltpu.async_copy(table_hbm.at[rows_v], buf_v, sem).wait()
            for g in range(chunk // 16):
                iv = idx_v[c, pl.ds(g * 16, 16)]
                offs = (iv & jnp.int32(ppr - 1)) * jnp.int32(d)
                for t in range(16):
                    s = g * 16 + t
                    v = buf_v[s, pl.ds(offs[t], d)]
                    outb_v[(s * d) // 128, pl.ds((s * d) % 128, d)] = v
            pltpu.sync_copy(
                outb_v, out_hbm.at[pl.ds((c0 + c) * orpc, orpc)])
            return 0

        lax.fori_loop(0, n_chunks, chunk_body, 0)

    return k(table8, idx2d)


def kernel(states_prev, log_weights_prev, observations, controls, A, B, C):
    n, m, d = states_prev.shape
    noise = _make_noise(n * m * d // 128)
    noise = noise.reshape(n, m, d)
    states_pred, lw = _run_predict(states_prev, noise, log_weights_prev,
                                   observations, controls, A, B, C)
    best_states = _run_stats(lw, states_pred)
    idx = _run_sample(lw)
    table8 = states_pred.reshape(n * m * d // 128, 128)
    idx2d = idx.reshape(n * m // 128, 128)
    states = _run_gather(table8, idx2d, d).reshape(n, m, d)
    log_weights = jnp.full((n, m), np.float32(-np.log(np.float32(m))),
                           jnp.float32)
    return best_states, states, log_weights


# speedup vs baseline: 4.9122x; 4.9122x over previous
import functools

import numpy as np
import jax
import jax.numpy as jnp
from jax import lax
from jax.experimental import pallas as pl
from jax.experimental.pallas import tpu as pltpu
from jax.experimental.pallas import tpu_sc as plsc


def _np_threefry2x32(k1, k2, x0, x1):
    u32 = np.uint32
    rotl = lambda x, r: u32((u32(x) << u32(r)) | (u32(x) >> u32(32 - r)))
    ks = [u32(k1), u32(k2), u32(u32(k1) ^ u32(k2) ^ u32(0x1BD11BDA))]
    x = [u32(x0 + ks[0]), u32(x1 + ks[1])]
    rots = ((13, 15, 26, 6), (17, 29, 16, 24))
    old = np.seterr(over="ignore")
    for i in range(5):
        for r in rots[i % 2]:
            x[0] = u32(x[0] + x[1])
            x[1] = u32(rotl(x[1], r) ^ x[0])
        x[0] = u32(x[0] + ks[(i + 1) % 3])
        x[1] = u32(x[1] + ks[(i + 2) % 3] + u32(i + 1))
    np.seterr(**old)
    return int(x[0]), int(x[1])


_BASE_K1, _BASE_K2 = 0, 42
_NOISE_KEY = _np_threefry2x32(_BASE_K1, _BASE_K2, 0, 0)
_CAT_KEY = _np_threefry2x32(_BASE_K1, _BASE_K2, 0, 1)

_TINY = np.float32(np.finfo(np.float32).tiny)
_U_LO = np.float32(np.nextafter(np.float32(-1.0), np.float32(0.0)))
_U_SPAN = np.float32(np.float32(1.0) - _U_LO)
_G_SPAN = np.float32(np.float32(1.0) - _TINY)
_SQRT2 = np.float32(np.sqrt(2.0))


def _tf2x32(k1, k2, x0, x1):
    ks0 = jnp.uint32(k1)
    ks1 = jnp.uint32(k2)
    ks2 = jnp.uint32(k1 ^ k2 ^ 0x1BD11BDA)
    kss = (ks0, ks1, ks2)

    def rounds(x0, x1, rs):
        for r in rs:
            x0 = x0 + x1
            x1 = (x1 << jnp.uint32(r)) | (x1 >> jnp.uint32(32 - r))
            x1 = x1 ^ x0
        return x0, x1

    rots = ((13, 15, 26, 6), (17, 29, 16, 24))
    x0 = x0 + ks0
    x1 = x1 + ks1
    for i in range(5):
        x0, x1 = rounds(x0, x1, rots[i % 2])
        x0 = x0 + kss[(i + 1) % 3]
        x1 = x1 + kss[(i + 2) % 3] + jnp.uint32(i + 1)
    return x0, x1


def _bits_to_unit(bits):
    fb = (bits >> jnp.uint32(9)) | jnp.uint32(0x3F800000)
    return lax.bitcast_convert_type(fb, jnp.float32) - jnp.float32(1.0)


def _erfinv_f32(x):
    w = -jnp.log(jnp.float32(1.0) - x * x)
    wc = w - jnp.float32(2.5)
    p = jnp.float32(2.81022636e-08)
    for c in (3.43273939e-07, -3.5233877e-06, -4.39150654e-06, 0.00021858087,
              -0.00125372503, -0.00417768164, 0.246640727, 1.50140941):
        p = jnp.float32(c) + p * wc
    wt = jnp.sqrt(w) - jnp.float32(3.0)
    q = jnp.float32(-0.000200214257)
    for c in (0.000100950558, 0.00134934322, -0.00367342844, 0.00573950773,
              -0.0076224613, 0.00943887047, 1.00167406, 2.83297682):
        q = jnp.float32(c) + q * wt
    return jnp.where(w < jnp.float32(5.0), p, q) * x


def _noise_body(out_ref):
    rb = pl.program_id(0)
    rows, lanes = out_ref.shape
    base = jnp.uint32(rb * rows * lanes)
    lo = (base
          + lax.broadcasted_iota(jnp.uint32, (rows, lanes), 0) * jnp.uint32(lanes)
          + lax.broadcasted_iota(jnp.uint32, (rows, lanes), 1))
    b0, b1 = _tf2x32(_NOISE_KEY[0], _NOISE_KEY[1], jnp.uint32(0), lo)
    f = _bits_to_unit(b0 ^ b1)
    u = jnp.maximum(_U_LO, f * _U_SPAN + _U_LO)
    out_ref[...] = (_SQRT2 * _erfinv_f32(u)) * jnp.float32(0.1)


def _make_noise(total_rows, lanes=128, block_rows=1024):
    block_rows = min(block_rows, total_rows)
    return pl.pallas_call(
        _noise_body,
        grid=(total_rows // block_rows,),
        out_specs=pl.BlockSpec((block_rows, lanes), lambda rb: (rb, 0)),
        out_shape=jax.ShapeDtypeStruct((total_rows, lanes), jnp.float32),
    )()


def _predict_body(sp_ref, noise_ref, lwp_ref, obs_ref, ctrl_ref, a_ref, b_ref,
                  c_ref, pred_ref, lw_ref):
    sp = sp_ref[0]
    cb = jnp.dot(ctrl_ref[0], b_ref[...],
                 preferred_element_type=jnp.float32)
    pred = (jnp.dot(sp, a_ref[...], preferred_element_type=jnp.float32)
            + cb + noise_ref[0])
    diff = jnp.dot(pred, c_ref[...],
                   preferred_element_type=jnp.float32) - obs_ref[0]
    meas = jnp.float32(-0.5) * jnp.sum(diff * diff, axis=1)
    pred_ref[0] = pred
    lw_ref[0, 0, :] = lwp_ref[0, 0] + meas


def _run_predict(states_prev, noise, log_weights_prev, observations, controls,
                 A, B, C, mb=2048):
    mb = min(mb, states_prev.shape[1])
    n, m, d = states_prev.shape
    do = observations.shape[1]
    dc = controls.shape[1]
    nb = m // mb
    lwp3 = log_weights_prev.reshape(n * nb, 1, mb)
    obs3 = observations.reshape(n, 1, do)
    ctrl3 = controls.reshape(n, 1, dc)
    pred, lw3 = pl.pallas_call(
        _predict_body,
        grid=(n, nb),
        in_specs=[
            pl.BlockSpec((1, mb, d), lambda j, b: (j, b, 0)),
            pl.BlockSpec((1, mb, d), lambda j, b: (j, b, 0)),
            pl.BlockSpec((1, 1, mb), lambda j, b, _nb=nb: (j * _nb + b, 0, 0)),
            pl.BlockSpec((1, 1, do), lambda j, b: (j, 0, 0)),
            pl.BlockSpec((1, 1, dc), lambda j, b: (j, 0, 0)),
            pl.BlockSpec((d, d), lambda j, b: (0, 0)),
            pl.BlockSpec((dc, d), lambda j, b: (0, 0)),
            pl.BlockSpec((d, do), lambda j, b: (0, 0)),
        ],
        out_specs=[
            pl.BlockSpec((1, mb, d), lambda j, b: (j, b, 0)),
            pl.BlockSpec((1, 1, mb), lambda j, b, _nb=nb: (j * _nb + b, 0, 0)),
        ],
        out_shape=[
            jax.ShapeDtypeStruct((n, m, d), jnp.float32),
            jax.ShapeDtypeStruct((n * nb, 1, mb), jnp.float32),
        ],
    )(states_prev, noise, lwp3, obs3, ctrl3, A, B, C)
    return pred, lw3.reshape(n, m)


def _stats_body(lw_ref, pred_ref, best_ref):
    lw = lw_ref[0]
    m0 = jnp.max(lw)
    e = jnp.exp(lw - m0)
    s = jnp.sum(e)
    acc = jnp.dot(e, pred_ref[0], preferred_element_type=jnp.float32)
    best_ref[0] = acc / s


def _run_stats(lw, states_pred):
    n, m, d = states_pred.shape
    best3 = pl.pallas_call(
        _stats_body,
        grid=(n,),
        in_specs=[
            pl.BlockSpec((1, 1, m), lambda j: (j, 0, 0)),
            pl.BlockSpec((1, m, d), lambda j: (j, 0, 0)),
        ],
        out_specs=pl.BlockSpec((1, 1, d), lambda j: (j, 0, 0)),
        out_shape=jax.ShapeDtypeStruct((n, 1, d), jnp.float32),
    )(lw.reshape(n, 1, m), states_pred)
    return best3.reshape(n, d)


def _sample_body(lw_ref, idx_ref, *, n, m, sb, ku=1024):
    j = pl.program_id(0)
    ib = pl.program_id(1)
    nm_shift = (n * m).bit_length() - 1
    m_shift = m.bit_length() - 1
    i_base = ib * sb
    hi = jnp.uint32(i_base >> (32 - nm_shift))
    lane_u = lax.broadcasted_iota(jnp.uint32, (8, ku), 1)
    lane_i = lax.broadcasted_iota(jnp.int32, (8, ku), 1)
    sub_u = lax.broadcasted_iota(jnp.uint32, (8, ku), 0)
    lo_ij = ((jnp.uint32(i_base) + sub_u) << jnp.uint32(nm_shift)) | (
        jnp.uint32(j) << jnp.uint32(m_shift))

    def isub_body(i_sub, _):
        lo0 = lo_ij + (jnp.uint32(i_sub * 8) << jnp.uint32(nm_shift))

        def kb_body(kb, carry):
            best_v, best_k = carry
            k0 = kb * ku
            lo = lo0 + jnp.uint32(k0) + lane_u
            b0, b1 = _tf2x32(_CAT_KEY[0], _CAT_KEY[1], hi, lo)
            f = _bits_to_unit(b0 ^ b1)
            u = jnp.maximum(_TINY, f * _G_SPAN + _TINY)
            g = -jnp.log(-jnp.log(u))
            lvec = lw_ref[0, 0, pl.ds(k0, ku)]
            tot = g + jnp.broadcast_to(lvec[None, :], (8, ku))
            kvec = jnp.int32(k0) + lane_i
            upd = tot > best_v
            return (jnp.where(upd, tot, best_v), jnp.where(upd, kvec, best_k))

        best_v, best_k = lax.fori_loop(
            0, m // ku, kb_body,
            (jnp.full((8, ku), -jnp.inf, jnp.float32),
             jnp.zeros((8, ku), jnp.int32)))
        vmax = jnp.max(best_v, axis=1, keepdims=True)
        kmin = jnp.min(jnp.where(best_v == vmax, best_k, jnp.int32(m)), axis=1)
        idx_ref[0, i_sub, :] = kmin + j * m
        return 0

    lax.fori_loop(0, sb // 8, isub_body, 0)


def _run_sample(lw, sb=256):
    n, m = lw.shape
    nib = m // sb
    body = functools.partial(_sample_body, n=n, m=m, sb=sb, ku=min(1024, m))
    idx3 = pl.pallas_call(
        body,
        grid=(n, nib),
        in_specs=[pl.BlockSpec((1, 1, m), lambda j, ib: (j, 0, 0))],
        out_specs=pl.BlockSpec((1, sb // 8, 8),
                               lambda j, ib, _nib=nib: (j * _nib + ib, 0, 0)),
        out_shape=jax.ShapeDtypeStruct((n * nib, sb // 8, 8), jnp.int32),
    )(lw.reshape(n, 1, m))
    return idx3.reshape(n, m)


def _run_gather(table8, idx2d, d, chunk=128):
    nrows, _ = table8.shape
    nm = nrows * (128 // d)
    ppr = 128 // d
    info = plsc.get_sparse_core_info()
    nw = info.num_cores * info.num_subcores
    n_chunks = nm // nw // chunk
    orpc = chunk * d // 128
    mesh = plsc.VectorSubcoreMesh(core_axis_name="c", subcore_axis_name="s")

    @functools.partial(
        pl.kernel,
        mesh=mesh,
        out_type=jax.ShapeDtypeStruct((nm * d // 128, 128), jnp.float32),
        scratch_types=[
            pltpu.VMEM((n_chunks, chunk), jnp.int32),
            pltpu.VMEM((chunk,), jnp.int32),
            pltpu.VMEM((chunk, 128), jnp.float32),
            pltpu.VMEM((orpc, 128), jnp.float32),
            pltpu.SemaphoreType.DMA,
        ],
    )
    def k(table_hbm, idx_hbm, out_hbm, idx_v, rows_v, buf_v, outb_v, sem):
        wid = lax.axis_index("s") * info.num_cores + lax.axis_index("c")
        c0 = wid * n_chunks
        pltpu.sync_copy(idx_hbm.at[pl.ds(c0, n_chunks)], idx_v)
        iota16 = lax.iota(jnp.int32, 16)

        def chunk_body(c, _):
            for g in range(chunk // 16):
                iv = idx_v[c, pl.ds(g * 16, 16)]
                rows_v[pl.ds(g * 16, 16)] = lax.shift_right_logical(iv, 3)
            pltpu.async_copy(table_hbm.at[rows_v], buf_v, sem).wait()
            for g in range(chunk // 16):
                iv = idx_v[c, pl.ds(g * 16, 16)]
                offs = (iv & jnp.int32(ppr - 1)) * jnp.int32(d)
                for t in range(16):
                    s = g * 16 + t
                    v = buf_v[s, pl.ds(offs[t], d)]
                    outb_v[(s * d) // 128, pl.ds((s * d) % 128, d)] = v
            pltpu.sync_copy(
                outb_v, out_hbm.at[pl.ds((c0 + c) * orpc, orpc)])
            return 0

        lax.fori_loop(0, n_chunks, chunk_body, 0)

    return k(table8, idx2d)


def kernel(states_prev, log_weights_prev, observations, controls, A, B, C):
    n, m, d = states_prev.shape
    noise = _make_noise(n * m * d // 128)
    noise = noise.reshape(n, m, d)
    states_pred, lw = _run_predict(states_prev, noise, log_weights_prev,
                                   observations, controls, A, B, C)
    best_states = _run_stats(lw, states_pred)
    idx = _run_sample(lw)
    table8 = states_pred.reshape(n * m * d // 128, 128)
    idx2d = idx.reshape(n * m // 128, 128)
    states = _run_gather(table8, idx2d, d).reshape(n, m, d)
    log_weights = jnp.full((n, m), np.float32(-np.log(np.float32(m))),
                           jnp.float32)
    return best_states, states, log_weights
